# Initial kernel scaffold; baseline (speedup 1.0000x reference)
import jax
import jax.numpy as jnp
from jax.experimental import pallas as pl


def kernel(x_paper, x_label, edge_index_pl, edge_index_lp, proj_paper_W, proj_paper_b, proj_label_W, proj_label_b, att_src_pl, att_dst_pl, att_src_lp, att_dst_lp, q, k_W, k_b, lin_W, lin_b):
    raise NotImplementedError("write your pallas kernel here")


# speedup vs baseline: 4.1030x; 4.1030x over previous
import functools

import jax
import jax.numpy as jnp
from jax import lax
from jax.experimental import pallas as pl
from jax.experimental.pallas import tpu as pltpu
from jax.experimental.pallas import tpu_sc as plsc

N = 10000
E = 160000
HID = 512
H = 8
D = 64
EMB = 256
NC = 2
NS = 16
BLK = 80
EPT = E // NS
NBLK = EPT // BLK
N_PAD = 10240
ROWS_PT = N_PAD // NS
RB = 1000
CHUNK = 128
NCHUNK = 4


def _k0_body(x_ref, w_ref, b_ref, a1_ref, a2_ref,
             o0_ref, o1_ref, o2_ref, o3_ref, s1_ref, s2_ref):
    acc = jnp.dot(x_ref[...], w_ref[...], preferred_element_type=jnp.float32)
    acc = acc + b_ref[...]
    outs = (o0_ref, o1_ref, o2_ref, o3_ref)
    for c in range(NCHUNK):
        outs[c][...] = acc[:, c * CHUNK:(c + 1) * CHUNK]
    s1_ref[...] = jnp.dot(acc, a1_ref[...], preferred_element_type=jnp.float32)
    s2_ref[...] = jnp.dot(acc, a2_ref[...], preferred_element_type=jnp.float32)


def _k0_call(x, w, b2, a1, a2):
    grid = (N // RB,)
    return pl.pallas_call(
        _k0_body,
        grid=grid,
        in_specs=[
            pl.BlockSpec((RB, HID), lambda i: (i, 0)),
            pl.BlockSpec((HID, HID), lambda i: (0, 0)),
            pl.BlockSpec((1, HID), lambda i: (0, 0)),
            pl.BlockSpec((HID, CHUNK), lambda i: (0, 0)),
            pl.BlockSpec((HID, CHUNK), lambda i: (0, 0)),
        ],
        out_specs=[pl.BlockSpec((RB, CHUNK), lambda i: (i, 0))] * (NCHUNK + 2),
        out_shape=[jax.ShapeDtypeStruct((N, CHUNK), jnp.float32)] * (NCHUNK + 2),
    )(x, w, b2, a1, a2)


def _k3_body(p_ref, den_ref, r_ref, w_ref, b_ref, o_ref):
    den = jnp.dot(den_ref[...], r_ref[...], preferred_element_type=jnp.float32)
    rec = 1.0 / (den + 1e-16)
    acc = jnp.zeros((RB, EMB), jnp.float32)
    for c in range(NCHUNK):
        u = jnp.maximum(p_ref[c] * rec[:, c * CHUNK:(c + 1) * CHUNK], 0.0)
        wc = w_ref[pl.ds(c * CHUNK, CHUNK), :]
        acc = acc + jnp.dot(u, wc, preferred_element_type=jnp.float32)
    o_ref[...] = acc + b_ref[...]


def _k3_call(p4, den, r, w, b2):
    grid = (N // RB,)
    return pl.pallas_call(
        _k3_body,
        grid=grid,
        in_specs=[
            pl.BlockSpec((NCHUNK, RB, CHUNK), lambda i: (0, i, 0)),
            pl.BlockSpec((RB, CHUNK), lambda i: (i, 0)),
            pl.BlockSpec((CHUNK, HID), lambda i: (0, 0)),
            pl.BlockSpec((HID, EMB), lambda i: (0, 0)),
            pl.BlockSpec((1, EMB), lambda i: (0, 0)),
        ],
        out_specs=pl.BlockSpec((RB, EMB), lambda i: (i, 0)),
        out_shape=jax.ShapeDtypeStruct((N, EMB), jnp.float32),
    )(p4, den, r, w, b2)


_MESH = plsc.VectorSubcoreMesh(core_axis_name="c", subcore_axis_name="s",
                               num_cores=NC, num_subcores=NS)


@functools.partial(
    pl.kernel,
    out_type=[
        jax.ShapeDtypeStruct((E * 16,), jnp.float32),
        jax.ShapeDtypeStruct((E * 16,), jnp.float32),
    ],
    mesh=_MESH,
    scratch_types=[
        pltpu.VMEM((EPT,), jnp.int32),
        pltpu.VMEM((EPT,), jnp.int32),
        pltpu.VMEM((BLK, CHUNK), jnp.float32),
        pltpu.VMEM((BLK, CHUNK), jnp.float32),
        pltpu.VMEM((BLK * 16,), jnp.float32),
        pltpu.SemaphoreType.DMA,
    ],
)
def _k1(a_src_pl, a_dst_pl, a_src_lp, a_dst_lp,
        src_pl, dst_pl, src_lp, dst_lp,
        alpha_pl, alpha_lp,
        src_v, dst_v, asrc_v, adst_v, exp_v, sem):
    tid = lax.axis_index("s")
    core = lax.axis_index("c")

    def run(a_src_t, a_dst_t, src1, dst1, alpha_out):
        pltpu.sync_copy(src1.at[pl.ds(tid * EPT, EPT)], src_v)
        pltpu.sync_copy(dst1.at[pl.ds(tid * EPT, EPT)], dst_v)

        def block(b, _):
            e0 = tid * EPT + b * BLK
            pltpu.async_copy(
                a_src_t.at[src_v.at[pl.ds(b * BLK, BLK)]], asrc_v, sem).wait()
            pltpu.async_copy(
                a_dst_t.at[dst_v.at[pl.ds(b * BLK, BLK)]], adst_v, sem).wait()

            def edge(j, _):
                v = asrc_v[j, pl.ds(0, 16)] + adst_v[j, pl.ds(0, 16)]
                v = jnp.where(v >= 0.0, v, 0.2 * v)
                exp_v[pl.ds(j * 16, 16)] = jnp.exp(v)
                return _
            lax.fori_loop(0, BLK, edge, None)
            pltpu.sync_copy(exp_v, alpha_out.at[pl.ds(e0 * 16, BLK * 16)])
            return _
        lax.fori_loop(0, NBLK, block, None)

    @pl.when(core == 0)
    def _():
        run(a_src_pl, a_dst_pl, src_pl, dst_pl, alpha_pl)

    @pl.when(core == 1)
    def _():
        run(a_src_lp, a_dst_lp, src_lp, dst_lp, alpha_lp)


@functools.partial(
    pl.kernel,
    out_type=[
        jax.ShapeDtypeStruct((NCHUNK, N_PAD, CHUNK), jnp.float32),
        jax.ShapeDtypeStruct((N_PAD, CHUNK), jnp.float32),
        jax.ShapeDtypeStruct((NCHUNK, N_PAD, CHUNK), jnp.float32),
        jax.ShapeDtypeStruct((N_PAD, CHUNK), jnp.float32),
    ],
    mesh=_MESH,
    scratch_types=[
        pltpu.VMEM_SHARED((N_PAD, CHUNK), jnp.float32),
        pltpu.VMEM((EPT,), jnp.int32),
        pltpu.VMEM((NBLK, BLK), jnp.int32),
        pltpu.VMEM((BLK * 16,), jnp.float32),
        pltpu.VMEM((BLK, CHUNK), jnp.float32),
        pltpu.SemaphoreType.DMA,
    ],
)
def _k2(xp0, xp1, xp2, xp3, xl0, xl1, xl2, xl3,
        src_pl, dst2_pl, src_lp, dst2_lp, alpha_pl, alpha_lp,
        out_label, den_pl, out_paper, den_lp,
        acc_sp, src_v, dst_v, alpha_v, rows_v, sem):
    tid = lax.axis_index("s")
    core = lax.axis_index("c")

    def zero_rows_v(j, _):
        for k in range(CHUNK // 16):
            rows_v[j, pl.ds(k * 16, 16)] = jnp.zeros((16,), jnp.float32)
        return _

    def clear_acc():
        lax.fori_loop(0, BLK, zero_rows_v, None)
        for q in range(ROWS_PT // BLK):
            pltpu.sync_copy(
                rows_v, acc_sp.at[pl.ds(tid * ROWS_PT + q * BLK, BLK)])
        plsc.subcore_barrier()

    def writeback(dst_hbm):
        plsc.subcore_barrier()
        for q in range(ROWS_PT // BLK):
            pltpu.sync_copy(
                acc_sp.at[pl.ds(tid * ROWS_PT + q * BLK, BLK)], rows_v)
            pltpu.sync_copy(rows_v, dst_hbm.at[pl.ds(q * BLK, BLK)])
        plsc.subcore_barrier()

    def run(xs_refs, src1, dst2, alpha_t, out_hbm, den_hbm):
        pltpu.sync_copy(src1.at[pl.ds(tid * EPT, EPT)], src_v)
        pltpu.sync_copy(dst2.at[tid], dst_v)
        for c in range(NCHUNK):
            clear_acc()

            def block(b, _):
                e0 = tid * EPT + b * BLK
                pltpu.async_copy(
                    xs_refs[c].at[src_v.at[pl.ds(b * BLK, BLK)]],
                    rows_v, sem).wait()
                pltpu.sync_copy(
                    alpha_t.at[pl.ds(e0 * 16, BLK * 16)], alpha_v)

                def edge(j, _):
                    av = alpha_v[pl.ds(j * 16, 16)]
                    w0 = av[2 * c]
                    w1 = av[2 * c + 1]
                    for k in range(4):
                        s = pl.ds(k * 16, 16)
                        rows_v[j, s] = rows_v[j, s] * w0
                    for k in range(4, 8):
                        s = pl.ds(k * 16, 16)
                        rows_v[j, s] = rows_v[j, s] * w1
                    return _
                lax.fori_loop(0, BLK, edge, None)
                pltpu.sync_copy(rows_v, acc_sp.at[dst_v.at[b]], add=True)
                return _
            lax.fori_loop(0, NBLK, block, None)
            writeback(out_hbm.at[c, pl.ds(tid * ROWS_PT, ROWS_PT)])

        clear_acc()

        def dblock(b, _):
            e0 = tid * EPT + b * BLK
            pltpu.sync_copy(alpha_t.at[pl.ds(e0 * 16, BLK * 16)], alpha_v)

            def edge(j, _):
                av = alpha_v[pl.ds(j * 16, 16)]
                for k in range(8):
                    s = pl.ds(k * 16, 16)
                    rows_v[j, s] = jnp.full((16,), av[k], jnp.float32)
                return _
            lax.fori_loop(0, BLK, edge, None)
            pltpu.sync_copy(rows_v, acc_sp.at[dst_v.at[b]], add=True)
            return _
        lax.fori_loop(0, NBLK, dblock, None)
        writeback(den_hbm.at[pl.ds(tid * ROWS_PT, ROWS_PT)])

    @pl.when(core == 0)
    def _():
        run((xp0, xp1, xp2, xp3), src_pl, dst2_pl, alpha_pl,
            out_label, den_pl)

    @pl.when(core == 1)
    def _():
        run((xl0, xl1, xl2, xl3), src_lp, dst2_lp, alpha_lp,
            out_paper, den_lp)


def _att_matrix(att):
    a = jnp.zeros((H, D, CHUNK), jnp.float32)
    a = a.at[jnp.arange(H), :, jnp.arange(H)].set(att)
    return a.reshape(HID, CHUNK)


def _head_expand_matrix():
    r = jnp.zeros((CHUNK, H, D), jnp.float32)
    r = r.at[16 * jnp.arange(H), jnp.arange(H), :].set(1.0)
    return r.reshape(CHUNK, HID)


def kernel(x_paper, x_label, edge_index_pl, edge_index_lp,
           proj_paper_W, proj_paper_b, proj_label_W, proj_label_b,
           att_src_pl, att_dst_pl, att_src_lp, att_dst_lp,
           q, k_W, k_b, lin_W, lin_b):
    del q, k_W, k_b

    a_src_pl_m = _att_matrix(att_src_pl)
    a_dst_pl_m = _att_matrix(att_dst_pl)
    a_src_lp_m = _att_matrix(att_src_lp)
    a_dst_lp_m = _att_matrix(att_dst_lp)
    r_m = _head_expand_matrix()

    bp2 = proj_paper_b.reshape(1, HID)
    bl2 = proj_label_b.reshape(1, HID)
    lb2 = lin_b.reshape(1, EMB)

    xp0, xp1, xp2, xp3, a_src_pl_t, a_dst_lp_t = _k0_call(
        x_paper, proj_paper_W, bp2, a_src_pl_m, a_dst_lp_m)
    xl0, xl1, xl2, xl3, a_dst_pl_t, a_src_lp_t = _k0_call(
        x_label, proj_label_W, bl2, a_dst_pl_m, a_src_lp_m)

    src_pl = edge_index_pl[0]
    dst_pl = edge_index_pl[1]
    src_lp = edge_index_lp[0]
    dst_lp = edge_index_lp[1]
    dst2_pl = dst_pl.reshape(NS, NBLK, BLK)
    dst2_lp = dst_lp.reshape(NS, NBLK, BLK)

    alpha_pl, alpha_lp = _k1(
        a_src_pl_t, a_dst_pl_t, a_src_lp_t, a_dst_lp_t,
        src_pl, dst_pl, src_lp, dst_lp)

    out_label4, den_pl, out_paper4, den_lp = _k2(
        xp0, xp1, xp2, xp3, xl0, xl1, xl2, xl3,
        src_pl, dst2_pl, src_lp, dst2_lp, alpha_pl, alpha_lp)

    h_paper = _k3_call(out_paper4, den_lp, r_m, lin_W, lb2)
    h_label = _k3_call(out_label4, den_pl, r_m, lin_W, lb2)
    return h_paper, h_label


# speedup vs baseline: 6.7116x; 1.6358x over previous
import functools

import jax
import jax.numpy as jnp
from jax import lax
from jax.experimental import pallas as pl
from jax.experimental.pallas import tpu as pltpu
from jax.experimental.pallas import tpu_sc as plsc

N = 10000
E = 160000
HID = 512
H = 8
D = 64
EMB = 256
NC = 2
NS = 16
BLK = 80
EPT = E // NS
NBLK = EPT // BLK
N_PAD = 10112
ROWS_PT = N_PAD // NS
WB_CHUNKS = ((0, 80), (80, 80), (160, 80), (240, 80), (320, 80),
             (400, 80), (480, 80), (560, 72))
RB = 1000
CHUNK = 128
NCHUNK = 4


def _k0_body(x_ref, w_ref, b_ref, a1_ref, a2_ref,
             o0_ref, o1_ref, o2_ref, o3_ref, s1_ref, s2_ref):
    acc = jnp.dot(x_ref[...], w_ref[...], preferred_element_type=jnp.float32)
    acc = acc + b_ref[...]
    outs = (o0_ref, o1_ref, o2_ref, o3_ref)
    for c in range(NCHUNK):
        outs[c][...] = acc[:, c * CHUNK:(c + 1) * CHUNK]
    s1_ref[...] = jnp.dot(acc, a1_ref[...], preferred_element_type=jnp.float32)
    s2_ref[...] = jnp.dot(acc, a2_ref[...], preferred_element_type=jnp.float32)


def _k0_call(x, w, b2, a1, a2):
    grid = (N // RB,)
    return pl.pallas_call(
        _k0_body,
        grid=grid,
        in_specs=[
            pl.BlockSpec((RB, HID), lambda i: (i, 0)),
            pl.BlockSpec((HID, HID), lambda i: (0, 0)),
            pl.BlockSpec((1, HID), lambda i: (0, 0)),
            pl.BlockSpec((HID, CHUNK), lambda i: (0, 0)),
            pl.BlockSpec((HID, CHUNK), lambda i: (0, 0)),
        ],
        out_specs=[pl.BlockSpec((RB, CHUNK), lambda i: (i, 0))] * (NCHUNK + 2),
        out_shape=[jax.ShapeDtypeStruct((N, CHUNK), jnp.float32)] * (NCHUNK + 2),
    )(x, w, b2, a1, a2)


def _k3_body(p_ref, den_ref, r_ref, w_ref, b_ref, o_ref):
    den = jnp.dot(den_ref[...], r_ref[...], preferred_element_type=jnp.float32)
    rec = 1.0 / (den + 1e-16)
    acc = jnp.zeros((RB, EMB), jnp.float32)
    for c in range(NCHUNK):
        u = jnp.maximum(p_ref[c] * rec[:, c * CHUNK:(c + 1) * CHUNK], 0.0)
        wc = w_ref[pl.ds(c * CHUNK, CHUNK), :]
        acc = acc + jnp.dot(u, wc, preferred_element_type=jnp.float32)
    o_ref[...] = acc + b_ref[...]


def _k3_call(p4, den, r, w, b2):
    grid = (N // RB,)
    return pl.pallas_call(
        _k3_body,
        grid=grid,
        in_specs=[
            pl.BlockSpec((NCHUNK, RB, CHUNK), lambda i: (0, i, 0)),
            pl.BlockSpec((RB, CHUNK), lambda i: (i, 0)),
            pl.BlockSpec((CHUNK, HID), lambda i: (0, 0)),
            pl.BlockSpec((HID, EMB), lambda i: (0, 0)),
            pl.BlockSpec((1, EMB), lambda i: (0, 0)),
        ],
        out_specs=pl.BlockSpec((RB, EMB), lambda i: (i, 0)),
        out_shape=jax.ShapeDtypeStruct((N, EMB), jnp.float32),
    )(p4, den, r, w, b2)


_MESH = plsc.VectorSubcoreMesh(core_axis_name="c", subcore_axis_name="s",
                               num_cores=NC, num_subcores=NS)


@functools.partial(
    pl.kernel,
    out_type=[
        jax.ShapeDtypeStruct((E * 16,), jnp.float32),
        jax.ShapeDtypeStruct((E * 16,), jnp.float32),
    ],
    mesh=_MESH,
    scratch_types=[
        pltpu.VMEM((EPT,), jnp.int32),
        pltpu.VMEM((EPT,), jnp.int32),
        pltpu.VMEM((BLK, CHUNK), jnp.float32),
        pltpu.VMEM((BLK, CHUNK), jnp.float32),
        pltpu.VMEM((BLK * 16,), jnp.float32),
        pltpu.SemaphoreType.DMA,
    ],
)
def _k1(a_src_pl, a_dst_pl, a_src_lp, a_dst_lp,
        src_pl, dst_pl, src_lp, dst_lp,
        alpha_pl, alpha_lp,
        src_v, dst_v, asrc_v, adst_v, exp_v, sem):
    tid = lax.axis_index("s")
    core = lax.axis_index("c")

    def run(a_src_t, a_dst_t, src1, dst1, alpha_out):
        pltpu.sync_copy(src1.at[pl.ds(tid * EPT, EPT)], src_v)
        pltpu.sync_copy(dst1.at[pl.ds(tid * EPT, EPT)], dst_v)

        def block(b, _):
            e0 = tid * EPT + b * BLK
            pltpu.async_copy(
                a_src_t.at[src_v.at[pl.ds(b * BLK, BLK)]], asrc_v, sem).wait()
            pltpu.async_copy(
                a_dst_t.at[dst_v.at[pl.ds(b * BLK, BLK)]], adst_v, sem).wait()

            def edge(j, _):
                v = asrc_v[j, pl.ds(0, 16)] + adst_v[j, pl.ds(0, 16)]
                v = jnp.where(v >= 0.0, v, 0.2 * v)
                exp_v[pl.ds(j * 16, 16)] = jnp.exp(v)
                return _
            lax.fori_loop(0, BLK, edge, None)
            pltpu.sync_copy(exp_v, alpha_out.at[pl.ds(e0 * 16, BLK * 16)])
            return _
        lax.fori_loop(0, NBLK, block, None)

    @pl.when(core == 0)
    def _():
        run(a_src_pl, a_dst_pl, src_pl, dst_pl, alpha_pl)

    @pl.when(core == 1)
    def _():
        run(a_src_lp, a_dst_lp, src_lp, dst_lp, alpha_lp)


@functools.partial(
    pl.kernel,
    out_type=[
        jax.ShapeDtypeStruct((NCHUNK, N_PAD, CHUNK), jnp.float32),
        jax.ShapeDtypeStruct((N_PAD, CHUNK), jnp.float32),
        jax.ShapeDtypeStruct((NCHUNK, N_PAD, CHUNK), jnp.float32),
        jax.ShapeDtypeStruct((N_PAD, CHUNK), jnp.float32),
    ],
    mesh=_MESH,
    scratch_types=[
        pltpu.VMEM_SHARED((N_PAD, CHUNK), jnp.float32),
        pltpu.VMEM((EPT,), jnp.int32),
        pltpu.VMEM((NBLK, BLK), jnp.int32),
        pltpu.VMEM((BLK * 16,), jnp.float32),
        pltpu.VMEM((BLK * 16,), jnp.float32),
        pltpu.VMEM((BLK, CHUNK), jnp.float32),
        pltpu.VMEM((BLK, CHUNK), jnp.float32),
        pltpu.SemaphoreType.DMA,
        pltpu.SemaphoreType.DMA,
    ],
)
def _k2(xp0, xp1, xp2, xp3, xl0, xl1, xl2, xl3,
        src_pl, dst2_pl, src_lp, dst2_lp, alpha_pl, alpha_lp,
        out_label, den_pl, out_paper, den_lp,
        acc_sp, src_v, dst_v, al0, al1, rows0, rows1, sem_r, sem_a):
    tid = lax.axis_index("s")
    core = lax.axis_index("c")
    rows = (rows0, rows1)
    als = (al0, al1)

    def zero_rows0(j, _):
        for k in range(CHUNK // 16):
            rows0[j, pl.ds(k * 16, 16)] = jnp.zeros((16,), jnp.float32)
        return _

    def clear_acc():
        lax.fori_loop(0, BLK, zero_rows0, None)
        for off, sz in WB_CHUNKS:
            pltpu.sync_copy(rows0.at[pl.ds(0, sz)],
                            acc_sp.at[pl.ds(tid * ROWS_PT + off, sz)])
        plsc.subcore_barrier()

    def writeback(dst_hbm):
        plsc.subcore_barrier()
        for off, sz in WB_CHUNKS:
            pltpu.sync_copy(acc_sp.at[pl.ds(tid * ROWS_PT + off, sz)],
                            rows0.at[pl.ds(0, sz)])
            pltpu.sync_copy(rows0.at[pl.ds(0, sz)], dst_hbm.at[pl.ds(off, sz)])
        plsc.subcore_barrier()

    def run(xs_refs, src1, dst2, alpha_t, out_hbm, den_hbm):
        pltpu.sync_copy(src1.at[pl.ds(tid * EPT, EPT)], src_v)
        pltpu.sync_copy(dst2.at[tid], dst_v)

        def gather_issue(c, b, buf):
            pltpu.async_copy(
                xs_refs[c].at[src_v.at[pl.ds(b * BLK, BLK)]], buf, sem_r)

        def alpha_issue(b, buf):
            e0 = tid * EPT + b * BLK
            pltpu.async_copy(
                alpha_t.at[pl.ds(e0 * 16, BLK * 16)], buf, sem_a)

        def drain_rows(buf):
            pltpu.make_async_copy(
                xs_refs[0].at[pl.ds(0, BLK)], buf, sem_r).wait()

        def drain_alpha(buf):
            pltpu.make_async_copy(
                alpha_t.at[pl.ds(0, BLK * 16)], buf, sem_a).wait()

        for c in range(NCHUNK):
            clear_acc()
            gather_issue(c, 0, rows0)
            alpha_issue(0, al0)
            gather_issue(c, 1, rows1)
            alpha_issue(1, al1)

            def scale_block(b, rbuf, abuf):
                def edge(j, _):
                    av = abuf[pl.ds(j * 16, 16)]
                    w0 = av[2 * c]
                    w1 = av[2 * c + 1]
                    for k in range(4):
                        s = pl.ds(k * 16, 16)
                        rbuf[j, s] = rbuf[j, s] * w0
                    for k in range(4, 8):
                        s = pl.ds(k * 16, 16)
                        rbuf[j, s] = rbuf[j, s] * w1
                    return _
                lax.fori_loop(0, BLK, edge, None)
                pltpu.sync_copy(rbuf, acc_sp.at[dst_v.at[b]], add=True)

            def pair(t, _):
                for k in range(2):
                    b = 2 * t + k
                    drain_rows(rows[k])
                    drain_alpha(als[k])
                    scale_block(b, rows[k], als[k])
                    nb = jnp.minimum(b + 2, NBLK - 1)
                    gather_issue(c, nb, rows[k])
                    alpha_issue(nb, als[k])
                return _
            lax.fori_loop(0, NBLK // 2, pair, None)
            drain_rows(rows0)
            drain_alpha(al0)
            scale_block(NBLK - 1, rows0, al0)
            drain_rows(rows1)
            drain_alpha(al1)
            writeback(out_hbm.at[c, pl.ds(tid * ROWS_PT, ROWS_PT)])

        clear_acc()
        alpha_issue(0, al0)
        alpha_issue(1, al1)

        def dblock(b, abuf):
            def edge(j, _):
                av = abuf[pl.ds(j * 16, 16)]
                for k in range(8):
                    s = pl.ds(k * 16, 16)
                    rows1[j, s] = jnp.full((16,), av[k], jnp.float32)
                return _
            lax.fori_loop(0, BLK, edge, None)
            pltpu.sync_copy(rows1, acc_sp.at[dst_v.at[b]], add=True)

        def dpair(t, _):
            for k in range(2):
                b = 2 * t + k
                drain_alpha(als[k])
                dblock(b, als[k])
                nb = jnp.minimum(b + 2, NBLK - 1)
                alpha_issue(nb, als[k])
            return _
        lax.fori_loop(0, NBLK // 2, dpair, None)
        drain_alpha(al0)
        dblock(NBLK - 1, al0)
        drain_alpha(al1)
        writeback(den_hbm.at[pl.ds(tid * ROWS_PT, ROWS_PT)])

    @pl.when(core == 0)
    def _():
        run((xp0, xp1, xp2, xp3), src_pl, dst2_pl, alpha_pl,
            out_label, den_pl)

    @pl.when(core == 1)
    def _():
        run((xl0, xl1, xl2, xl3), src_lp, dst2_lp, alpha_lp,
            out_paper, den_lp)


def _att_matrix(att):
    a = jnp.zeros((H, D, CHUNK), jnp.float32)
    a = a.at[jnp.arange(H), :, jnp.arange(H)].set(att)
    return a.reshape(HID, CHUNK)


def _head_expand_matrix():
    r = jnp.zeros((CHUNK, H, D), jnp.float32)
    r = r.at[16 * jnp.arange(H), jnp.arange(H), :].set(1.0)
    return r.reshape(CHUNK, HID)


def kernel(x_paper, x_label, edge_index_pl, edge_index_lp,
           proj_paper_W, proj_paper_b, proj_label_W, proj_label_b,
           att_src_pl, att_dst_pl, att_src_lp, att_dst_lp,
           q, k_W, k_b, lin_W, lin_b):
    del q, k_W, k_b

    a_src_pl_m = _att_matrix(att_src_pl)
    a_dst_pl_m = _att_matrix(att_dst_pl)
    a_src_lp_m = _att_matrix(att_src_lp)
    a_dst_lp_m = _att_matrix(att_dst_lp)
    r_m = _head_expand_matrix()

    bp2 = proj_paper_b.reshape(1, HID)
    bl2 = proj_label_b.reshape(1, HID)
    lb2 = lin_b.reshape(1, EMB)

    xp0, xp1, xp2, xp3, a_src_pl_t, a_dst_lp_t = _k0_call(
        x_paper, proj_paper_W, bp2, a_src_pl_m, a_dst_lp_m)
    xl0, xl1, xl2, xl3, a_dst_pl_t, a_src_lp_t = _k0_call(
        x_label, proj_label_W, bl2, a_dst_pl_m, a_src_lp_m)

    src_pl = edge_index_pl[0]
    dst_pl = edge_index_pl[1]
    src_lp = edge_index_lp[0]
    dst_lp = edge_index_lp[1]
    dst2_pl = dst_pl.reshape(NS, NBLK, BLK)
    dst2_lp = dst_lp.reshape(NS, NBLK, BLK)

    alpha_pl, alpha_lp = _k1(
        a_src_pl_t, a_dst_pl_t, a_src_lp_t, a_dst_lp_t,
        src_pl, dst_pl, src_lp, dst_lp)

    out_label4, den_pl, out_paper4, den_lp = _k2(
        xp0, xp1, xp2, xp3, xl0, xl1, xl2, xl3,
        src_pl, dst2_pl, src_lp, dst2_lp, alpha_pl, alpha_lp)

    h_paper = _k3_call(out_paper4, den_lp, r_m, lin_W, lb2)
    h_label = _k3_call(out_label4, den_pl, r_m, lin_W, lb2)
    return h_paper, h_label


# speedup vs baseline: 8.2107x; 1.2234x over previous
import functools

import jax
import jax.numpy as jnp
from jax import lax
from jax.experimental import pallas as pl
from jax.experimental.pallas import tpu as pltpu
from jax.experimental.pallas import tpu_sc as plsc

N = 10000
E = 160000
HID = 512
H = 8
D = 64
EMB = 256
NC = 2
NS = 16
BLK = 80
EPT = E // NS
NBLK = EPT // BLK
N_PAD = 10112
ROWS_PT = N_PAD // NS
WB_CHUNKS = ((0, 80), (80, 80), (160, 80), (240, 80), (320, 80),
             (400, 80), (480, 80), (560, 72))
RB = 1000
CHUNK = 128
NCHUNK = 4


def _k0_body(x_ref, w_ref, b_ref, a1_ref, a2_ref,
             o0_ref, o1_ref, o2_ref, o3_ref, s1_ref, s2_ref):
    acc = jnp.dot(x_ref[...], w_ref[...], preferred_element_type=jnp.float32)
    acc = acc + b_ref[...]
    outs = (o0_ref, o1_ref, o2_ref, o3_ref)
    for c in range(NCHUNK):
        outs[c][...] = acc[:, c * CHUNK:(c + 1) * CHUNK]
    s1_ref[...] = jnp.dot(acc, a1_ref[...], preferred_element_type=jnp.float32)
    s2_ref[...] = jnp.dot(acc, a2_ref[...], preferred_element_type=jnp.float32)


def _k0_call(x, w, b2, a1, a2):
    grid = (N // RB,)
    return pl.pallas_call(
        _k0_body,
        grid=grid,
        in_specs=[
            pl.BlockSpec((RB, HID), lambda i: (i, 0)),
            pl.BlockSpec((HID, HID), lambda i: (0, 0)),
            pl.BlockSpec((1, HID), lambda i: (0, 0)),
            pl.BlockSpec((HID, CHUNK), lambda i: (0, 0)),
            pl.BlockSpec((HID, CHUNK), lambda i: (0, 0)),
        ],
        out_specs=[pl.BlockSpec((RB, CHUNK), lambda i: (i, 0))] * (NCHUNK + 2),
        out_shape=[jax.ShapeDtypeStruct((N, CHUNK), jnp.float32)] * (NCHUNK + 2),
    )(x, w, b2, a1, a2)


def _k3_body(p_ref, den_ref, r_ref, w_ref, b_ref, o_ref):
    den = jnp.dot(den_ref[...], r_ref[...], preferred_element_type=jnp.float32)
    rec = 1.0 / (den + 1e-16)
    acc = jnp.zeros((RB, EMB), jnp.float32)
    for c in range(NCHUNK):
        u = jnp.maximum(p_ref[c] * rec[:, c * CHUNK:(c + 1) * CHUNK], 0.0)
        wc = w_ref[pl.ds(c * CHUNK, CHUNK), :]
        acc = acc + jnp.dot(u, wc, preferred_element_type=jnp.float32)
    o_ref[...] = acc + b_ref[...]


def _k3_call(p4, den, r, w, b2):
    grid = (N // RB,)
    return pl.pallas_call(
        _k3_body,
        grid=grid,
        in_specs=[
            pl.BlockSpec((NCHUNK, RB, CHUNK), lambda i: (0, i, 0)),
            pl.BlockSpec((RB, CHUNK), lambda i: (i, 0)),
            pl.BlockSpec((CHUNK, HID), lambda i: (0, 0)),
            pl.BlockSpec((HID, EMB), lambda i: (0, 0)),
            pl.BlockSpec((1, EMB), lambda i: (0, 0)),
        ],
        out_specs=pl.BlockSpec((RB, EMB), lambda i: (i, 0)),
        out_shape=jax.ShapeDtypeStruct((N, EMB), jnp.float32),
    )(p4, den, r, w, b2)


_MESH = plsc.VectorSubcoreMesh(core_axis_name="c", subcore_axis_name="s",
                               num_cores=NC, num_subcores=NS)


@functools.partial(
    pl.kernel,
    out_type=[
        jax.ShapeDtypeStruct((E * 16,), jnp.float32),
        jax.ShapeDtypeStruct((E * 16,), jnp.float32),
    ],
    mesh=_MESH,
    scratch_types=[
        pltpu.VMEM((EPT,), jnp.int32),
        pltpu.VMEM((EPT,), jnp.int32),
        pltpu.VMEM((BLK, CHUNK), jnp.float32),
        pltpu.VMEM((BLK, CHUNK), jnp.float32),
        pltpu.VMEM((BLK, CHUNK), jnp.float32),
        pltpu.VMEM((BLK, CHUNK), jnp.float32),
        pltpu.VMEM((BLK * 16,), jnp.float32),
        pltpu.VMEM((BLK * 16,), jnp.float32),
        pltpu.SemaphoreType.DMA,
        pltpu.SemaphoreType.DMA,
        pltpu.SemaphoreType.DMA,
    ],
)
def _k1(a_src_pl, a_dst_pl, a_src_lp, a_dst_lp,
        src_pl, dst_pl, src_lp, dst_lp,
        alpha_pl, alpha_lp,
        src_v, dst_v, as0, as1, ad0, ad1, ex0, ex1, sem_s, sem_d, sem_e):
    tid = lax.axis_index("s")
    core = lax.axis_index("c")
    asrcs = (as0, as1)
    adsts = (ad0, ad1)
    exps = (ex0, ex1)

    def run(a_src_t, a_dst_t, src1, dst1, alpha_out):
        pltpu.sync_copy(src1.at[pl.ds(tid * EPT, EPT)], src_v)
        pltpu.sync_copy(dst1.at[pl.ds(tid * EPT, EPT)], dst_v)

        def issue(b, sbuf, dbuf):
            pltpu.async_copy(
                a_src_t.at[src_v.at[pl.ds(b * BLK, BLK)]], sbuf, sem_s)
            pltpu.async_copy(
                a_dst_t.at[dst_v.at[pl.ds(b * BLK, BLK)]], dbuf, sem_d)

        def drain_gathers(sbuf, dbuf):
            pltpu.make_async_copy(
                a_src_t.at[pl.ds(0, BLK)], sbuf, sem_s).wait()
            pltpu.make_async_copy(
                a_dst_t.at[pl.ds(0, BLK)], dbuf, sem_d).wait()

        def drain_store():
            pltpu.make_async_copy(
                alpha_out.at[pl.ds(0, BLK * 16)], ex0, sem_e).wait()

        def proc(b, sbuf, dbuf, ebuf):
            def edge(j, _):
                v = sbuf[j, pl.ds(0, 16)] + dbuf[j, pl.ds(0, 16)]
                v = jnp.where(v >= 0.0, v, 0.2 * v)
                ebuf[pl.ds(j * 16, 16)] = jnp.exp(v)
                return _
            lax.fori_loop(0, BLK, edge, None)
            e0 = tid * EPT + b * BLK
            pltpu.async_copy(
                ebuf, alpha_out.at[pl.ds(e0 * 16, BLK * 16)], sem_e)

        issue(0, as0, ad0)
        issue(1, as1, ad1)

        def pair(t, _):
            for k in range(2):
                b = 2 * t + k
                drain_gathers(asrcs[k], adsts[k])

                @pl.when(t > 0)
                def _():
                    drain_store()
                proc(b, asrcs[k], adsts[k], exps[k])
                nb = jnp.minimum(b + 2, NBLK - 1)
                issue(nb, asrcs[k], adsts[k])
            return _
        lax.fori_loop(0, NBLK // 2, pair, None)
        drain_gathers(as0, ad0)
        drain_store()
        proc(NBLK - 1, as0, ad0, ex0)
        drain_gathers(as1, ad1)
        drain_store()
        drain_store()

    @pl.when(core == 0)
    def _():
        run(a_src_pl, a_dst_pl, src_pl, dst_pl, alpha_pl)

    @pl.when(core == 1)
    def _():
        run(a_src_lp, a_dst_lp, src_lp, dst_lp, alpha_lp)


@functools.partial(
    pl.kernel,
    out_type=[
        jax.ShapeDtypeStruct((NCHUNK, N_PAD, CHUNK), jnp.float32),
        jax.ShapeDtypeStruct((N_PAD, CHUNK), jnp.float32),
        jax.ShapeDtypeStruct((NCHUNK, N_PAD, CHUNK), jnp.float32),
        jax.ShapeDtypeStruct((N_PAD, CHUNK), jnp.float32),
    ],
    mesh=_MESH,
    scratch_types=[
        pltpu.VMEM_SHARED((N_PAD, CHUNK), jnp.float32),
        pltpu.VMEM((EPT,), jnp.int32),
        pltpu.VMEM((NBLK, BLK), jnp.int32),
        pltpu.VMEM((BLK * 16,), jnp.float32),
        pltpu.VMEM((BLK * 16,), jnp.float32),
        pltpu.VMEM((BLK, CHUNK), jnp.float32),
        pltpu.VMEM((BLK, CHUNK), jnp.float32),
        pltpu.SemaphoreType.DMA,
        pltpu.SemaphoreType.DMA,
    ],
)
def _k2(xp0, xp1, xp2, xp3, xl0, xl1, xl2, xl3,
        src_pl, dst2_pl, src_lp, dst2_lp, alpha_pl, alpha_lp,
        out_label, den_pl, out_paper, den_lp,
        acc_sp, src_v, dst_v, al0, al1, rows0, rows1, sem_r, sem_a):
    tid = lax.axis_index("s")
    core = lax.axis_index("c")
    rows = (rows0, rows1)
    als = (al0, al1)

    def zero_rows0(j, _):
        for k in range(CHUNK // 16):
            rows0[j, pl.ds(k * 16, 16)] = jnp.zeros((16,), jnp.float32)
        return _

    def clear_acc():
        lax.fori_loop(0, BLK, zero_rows0, None)
        for off, sz in WB_CHUNKS:
            pltpu.sync_copy(rows0.at[pl.ds(0, sz)],
                            acc_sp.at[pl.ds(tid * ROWS_PT + off, sz)])
        plsc.subcore_barrier()

    def writeback(dst_hbm):
        plsc.subcore_barrier()
        for off, sz in WB_CHUNKS:
            pltpu.sync_copy(acc_sp.at[pl.ds(tid * ROWS_PT + off, sz)],
                            rows0.at[pl.ds(0, sz)])
            pltpu.sync_copy(rows0.at[pl.ds(0, sz)], dst_hbm.at[pl.ds(off, sz)])
        plsc.subcore_barrier()

    def run(xs_refs, src1, dst2, alpha_t, out_hbm, den_hbm):
        pltpu.sync_copy(src1.at[pl.ds(tid * EPT, EPT)], src_v)
        pltpu.sync_copy(dst2.at[tid], dst_v)

        def gather_issue(c, b, buf):
            pltpu.async_copy(
                xs_refs[c].at[src_v.at[pl.ds(b * BLK, BLK)]], buf, sem_r)

        def alpha_issue(b, buf):
            e0 = tid * EPT + b * BLK
            pltpu.async_copy(
                alpha_t.at[pl.ds(e0 * 16, BLK * 16)], buf, sem_a)

        def drain_rows(buf):
            pltpu.make_async_copy(
                xs_refs[0].at[pl.ds(0, BLK)], buf, sem_r).wait()

        def drain_alpha(buf):
            pltpu.make_async_copy(
                alpha_t.at[pl.ds(0, BLK * 16)], buf, sem_a).wait()

        for c in range(NCHUNK):
            clear_acc()
            gather_issue(c, 0, rows0)
            alpha_issue(0, al0)
            gather_issue(c, 1, rows1)
            alpha_issue(1, al1)

            def scale_block(b, rbuf, abuf):
                def edge(j, _):
                    av = abuf[pl.ds(j * 16, 16)]
                    w0 = av[2 * c]
                    w1 = av[2 * c + 1]
                    for k in range(4):
                        s = pl.ds(k * 16, 16)
                        rbuf[j, s] = rbuf[j, s] * w0
                    for k in range(4, 8):
                        s = pl.ds(k * 16, 16)
                        rbuf[j, s] = rbuf[j, s] * w1
                    return _
                lax.fori_loop(0, BLK, edge, None)
                pltpu.sync_copy(rbuf, acc_sp.at[dst_v.at[b]], add=True)

            def pair(t, _):
                for k in range(2):
                    b = 2 * t + k
                    drain_rows(rows[k])
                    drain_alpha(als[k])
                    scale_block(b, rows[k], als[k])
                    nb = jnp.minimum(b + 2, NBLK - 1)
                    gather_issue(c, nb, rows[k])
                    alpha_issue(nb, als[k])
                return _
            lax.fori_loop(0, NBLK // 2, pair, None)
            drain_rows(rows0)
            drain_alpha(al0)
            scale_block(NBLK - 1, rows0, al0)
            drain_rows(rows1)
            drain_alpha(al1)
            writeback(out_hbm.at[c, pl.ds(tid * ROWS_PT, ROWS_PT)])

        clear_acc()
        alpha_issue(0, al0)
        alpha_issue(1, al1)

        def dblock(b, abuf):
            def edge(j, _):
                av = abuf[pl.ds(j * 16, 16)]
                for k in range(8):
                    s = pl.ds(k * 16, 16)
                    rows1[j, s] = jnp.full((16,), av[k], jnp.float32)
                return _
            lax.fori_loop(0, BLK, edge, None)
            pltpu.sync_copy(rows1, acc_sp.at[dst_v.at[b]], add=True)

        def dpair(t, _):
            for k in range(2):
                b = 2 * t + k
                drain_alpha(als[k])
                dblock(b, als[k])
                nb = jnp.minimum(b + 2, NBLK - 1)
                alpha_issue(nb, als[k])
            return _
        lax.fori_loop(0, NBLK // 2, dpair, None)
        drain_alpha(al0)
        dblock(NBLK - 1, al0)
        drain_alpha(al1)
        writeback(den_hbm.at[pl.ds(tid * ROWS_PT, ROWS_PT)])

    @pl.when(core == 0)
    def _():
        run((xp0, xp1, xp2, xp3), src_pl, dst2_pl, alpha_pl,
            out_label, den_pl)

    @pl.when(core == 1)
    def _():
        run((xl0, xl1, xl2, xl3), src_lp, dst2_lp, alpha_lp,
            out_paper, den_lp)


def _att_matrix(att):
    a = jnp.zeros((H, D, CHUNK), jnp.float32)
    a = a.at[jnp.arange(H), :, jnp.arange(H)].set(att)
    return a.reshape(HID, CHUNK)


def _head_expand_matrix():
    r = jnp.zeros((CHUNK, H, D), jnp.float32)
    r = r.at[16 * jnp.arange(H), jnp.arange(H), :].set(1.0)
    return r.reshape(CHUNK, HID)


def kernel(x_paper, x_label, edge_index_pl, edge_index_lp,
           proj_paper_W, proj_paper_b, proj_label_W, proj_label_b,
           att_src_pl, att_dst_pl, att_src_lp, att_dst_lp,
           q, k_W, k_b, lin_W, lin_b):
    del q, k_W, k_b

    a_src_pl_m = _att_matrix(att_src_pl)
    a_dst_pl_m = _att_matrix(att_dst_pl)
    a_src_lp_m = _att_matrix(att_src_lp)
    a_dst_lp_m = _att_matrix(att_dst_lp)
    r_m = _head_expand_matrix()

    bp2 = proj_paper_b.reshape(1, HID)
    bl2 = proj_label_b.reshape(1, HID)
    lb2 = lin_b.reshape(1, EMB)

    xp0, xp1, xp2, xp3, a_src_pl_t, a_dst_lp_t = _k0_call(
        x_paper, proj_paper_W, bp2, a_src_pl_m, a_dst_lp_m)
    xl0, xl1, xl2, xl3, a_dst_pl_t, a_src_lp_t = _k0_call(
        x_label, proj_label_W, bl2, a_dst_pl_m, a_src_lp_m)

    src_pl = edge_index_pl[0]
    dst_pl = edge_index_pl[1]
    src_lp = edge_index_lp[0]
    dst_lp = edge_index_lp[1]
    dst2_pl = dst_pl.reshape(NS, NBLK, BLK)
    dst2_lp = dst_lp.reshape(NS, NBLK, BLK)

    alpha_pl, alpha_lp = _k1(
        a_src_pl_t, a_dst_pl_t, a_src_lp_t, a_dst_lp_t,
        src_pl, dst_pl, src_lp, dst_lp)

    out_label4, den_pl, out_paper4, den_lp = _k2(
        xp0, xp1, xp2, xp3, xl0, xl1, xl2, xl3,
        src_pl, dst2_pl, src_lp, dst2_lp, alpha_pl, alpha_lp)

    h_paper = _k3_call(out_paper4, den_lp, r_m, lin_W, lb2)
    h_label = _k3_call(out_label4, den_pl, r_m, lin_W, lb2)
    return h_paper, h_label


# speedup vs baseline: 8.5071x; 1.0361x over previous
import functools

import jax
import jax.numpy as jnp
from jax import lax
from jax.experimental import pallas as pl
from jax.experimental.pallas import tpu as pltpu
from jax.experimental.pallas import tpu_sc as plsc

N = 10000
E = 160000
HID = 512
H = 8
D = 64
EMB = 256
NC = 2
NS = 16
BLK = 80
EPT = E // NS
NBLK = EPT // BLK
N_PAD = 10112
ROWS_PT = N_PAD // NS
WB_CHUNKS = ((0, 80), (80, 80), (160, 80), (240, 80), (320, 80),
             (400, 80), (480, 80), (560, 72))
RB = 1000
CHUNK = 128
NCHUNK = 4


def _k0_body(x_ref, w_ref, b_ref, a1_ref, a2_ref,
             o0_ref, o1_ref, o2_ref, o3_ref, s1_ref, s2_ref):
    acc = jnp.dot(x_ref[...], w_ref[...], preferred_element_type=jnp.float32)
    acc = acc + b_ref[...]
    outs = (o0_ref, o1_ref, o2_ref, o3_ref)
    for c in range(NCHUNK):
        outs[c][...] = acc[:, c * CHUNK:(c + 1) * CHUNK]
    s1_ref[...] = jnp.dot(acc, a1_ref[...], preferred_element_type=jnp.float32)
    s2_ref[...] = jnp.dot(acc, a2_ref[...], preferred_element_type=jnp.float32)


def _k0_call(x, w, b2, a1, a2):
    grid = (N // RB,)
    return pl.pallas_call(
        _k0_body,
        grid=grid,
        in_specs=[
            pl.BlockSpec((RB, HID), lambda i: (i, 0)),
            pl.BlockSpec((HID, HID), lambda i: (0, 0)),
            pl.BlockSpec((1, HID), lambda i: (0, 0)),
            pl.BlockSpec((HID, CHUNK), lambda i: (0, 0)),
            pl.BlockSpec((HID, CHUNK), lambda i: (0, 0)),
        ],
        out_specs=[pl.BlockSpec((RB, CHUNK), lambda i: (i, 0))] * (NCHUNK + 2),
        out_shape=[jax.ShapeDtypeStruct((N, CHUNK), jnp.float32)] * (NCHUNK + 2),
    )(x, w, b2, a1, a2)


def _k3_body(p_ref, den_ref, r_ref, w_ref, b_ref, o_ref):
    den = jnp.dot(den_ref[...], r_ref[...], preferred_element_type=jnp.float32)
    rec = 1.0 / (den + 1e-16)
    acc = jnp.zeros((RB, EMB), jnp.float32)
    for c in range(NCHUNK):
        u = jnp.maximum(p_ref[c] * rec[:, c * CHUNK:(c + 1) * CHUNK], 0.0)
        wc = w_ref[pl.ds(c * CHUNK, CHUNK), :]
        acc = acc + jnp.dot(u, wc, preferred_element_type=jnp.float32)
    o_ref[...] = acc + b_ref[...]


def _k3_call(p4, den, r, w, b2):
    grid = (N // RB,)
    return pl.pallas_call(
        _k3_body,
        grid=grid,
        in_specs=[
            pl.BlockSpec((NCHUNK, RB, CHUNK), lambda i: (0, i, 0)),
            pl.BlockSpec((RB, CHUNK), lambda i: (i, 0)),
            pl.BlockSpec((CHUNK, HID), lambda i: (0, 0)),
            pl.BlockSpec((HID, EMB), lambda i: (0, 0)),
            pl.BlockSpec((1, EMB), lambda i: (0, 0)),
        ],
        out_specs=pl.BlockSpec((RB, EMB), lambda i: (i, 0)),
        out_shape=jax.ShapeDtypeStruct((N, EMB), jnp.float32),
    )(p4, den, r, w, b2)


_MESH = plsc.VectorSubcoreMesh(core_axis_name="c", subcore_axis_name="s",
                               num_cores=NC, num_subcores=NS)


@functools.partial(
    pl.kernel,
    out_type=[
        jax.ShapeDtypeStruct((E * 16,), jnp.float32),
        jax.ShapeDtypeStruct((E * 16,), jnp.float32),
    ],
    mesh=_MESH,
    scratch_types=[
        pltpu.VMEM((EPT,), jnp.int32),
        pltpu.VMEM((EPT,), jnp.int32),
        pltpu.VMEM((BLK, CHUNK), jnp.float32),
        pltpu.VMEM((BLK, CHUNK), jnp.float32),
        pltpu.VMEM((BLK, CHUNK), jnp.float32),
        pltpu.VMEM((BLK, CHUNK), jnp.float32),
        pltpu.VMEM((BLK * 16,), jnp.float32),
        pltpu.VMEM((BLK * 16,), jnp.float32),
        pltpu.SemaphoreType.DMA,
        pltpu.SemaphoreType.DMA,
        pltpu.SemaphoreType.DMA,
    ],
)
def _k1(a_src_pl, a_dst_pl, a_src_lp, a_dst_lp,
        src_pl, dst_pl, src_lp, dst_lp,
        alpha_pl, alpha_lp,
        src_v, dst_v, as0, as1, ad0, ad1, ex0, ex1, sem_s, sem_d, sem_e):
    tid = lax.axis_index("s")
    core = lax.axis_index("c")
    asrcs = (as0, as1)
    adsts = (ad0, ad1)
    exps = (ex0, ex1)

    def run(a_src_t, a_dst_t, src1, dst1, alpha_out):
        pltpu.sync_copy(src1.at[pl.ds(tid * EPT, EPT)], src_v)
        pltpu.sync_copy(dst1.at[pl.ds(tid * EPT, EPT)], dst_v)

        def issue(b, sbuf, dbuf):
            pltpu.async_copy(
                a_src_t.at[src_v.at[pl.ds(b * BLK, BLK)]], sbuf, sem_s)
            pltpu.async_copy(
                a_dst_t.at[dst_v.at[pl.ds(b * BLK, BLK)]], dbuf, sem_d)

        def drain_gathers(sbuf, dbuf):
            pltpu.make_async_copy(
                a_src_t.at[pl.ds(0, BLK)], sbuf, sem_s).wait()
            pltpu.make_async_copy(
                a_dst_t.at[pl.ds(0, BLK)], dbuf, sem_d).wait()

        def drain_store():
            pltpu.make_async_copy(
                alpha_out.at[pl.ds(0, BLK * 16)], ex0, sem_e).wait()

        def proc(b, sbuf, dbuf, ebuf):
            def edge(j, _):
                v = sbuf[j, pl.ds(0, 16)] + dbuf[j, pl.ds(0, 16)]
                v = jnp.where(v >= 0.0, v, 0.2 * v)
                ebuf[pl.ds(j * 16, 16)] = jnp.exp(v)
                return _
            lax.fori_loop(0, BLK, edge, None)
            e0 = tid * EPT + b * BLK
            pltpu.async_copy(
                ebuf, alpha_out.at[pl.ds(e0 * 16, BLK * 16)], sem_e)

        issue(0, as0, ad0)
        issue(1, as1, ad1)

        def pair(t, _):
            for k in range(2):
                b = 2 * t + k
                drain_gathers(asrcs[k], adsts[k])

                @pl.when(t > 0)
                def _():
                    drain_store()
                proc(b, asrcs[k], adsts[k], exps[k])
                nb = jnp.minimum(b + 2, NBLK - 1)
                issue(nb, asrcs[k], adsts[k])
            return _
        lax.fori_loop(0, NBLK // 2, pair, None)
        drain_gathers(as0, ad0)
        drain_store()
        proc(NBLK - 1, as0, ad0, ex0)
        drain_gathers(as1, ad1)
        drain_store()
        drain_store()

    @pl.when(core == 0)
    def _():
        run(a_src_pl, a_dst_pl, src_pl, dst_pl, alpha_pl)

    @pl.when(core == 1)
    def _():
        run(a_src_lp, a_dst_lp, src_lp, dst_lp, alpha_lp)


@functools.partial(
    pl.kernel,
    out_type=[
        jax.ShapeDtypeStruct((NCHUNK, N_PAD, CHUNK), jnp.float32),
        jax.ShapeDtypeStruct((N_PAD, CHUNK), jnp.float32),
        jax.ShapeDtypeStruct((NCHUNK, N_PAD, CHUNK), jnp.float32),
        jax.ShapeDtypeStruct((N_PAD, CHUNK), jnp.float32),
    ],
    mesh=_MESH,
    scratch_types=[
        pltpu.VMEM_SHARED((N_PAD, CHUNK), jnp.float32),
        pltpu.VMEM((EPT,), jnp.int32),
        pltpu.VMEM((NBLK, BLK), jnp.int32),
        pltpu.VMEM((BLK * 16,), jnp.float32),
        pltpu.VMEM((BLK * 16,), jnp.float32),
        pltpu.VMEM((BLK, CHUNK), jnp.float32),
        pltpu.VMEM((BLK, CHUNK), jnp.float32),
        pltpu.SemaphoreType.DMA,
        pltpu.SemaphoreType.DMA,
    ],
)
def _k2(xp0, xp1, xp2, xp3, xl0, xl1, xl2, xl3,
        src_pl, dst2_pl, src_lp, dst2_lp, alpha_pl, alpha_lp,
        out_label, den_pl, out_paper, den_lp,
        acc_sp, src_v, dst_v, al0, al1, rows0, rows1, sem_r, sem_a):
    tid = lax.axis_index("s")
    core = lax.axis_index("c")
    rows = (rows0, rows1)
    als = (al0, al1)

    def zero_rows0(j, _):
        for k in range(CHUNK // 16):
            rows0[j, pl.ds(k * 16, 16)] = jnp.zeros((16,), jnp.float32)
        return _

    def clear_acc():
        lax.fori_loop(0, BLK, zero_rows0, None)
        for off, sz in WB_CHUNKS:
            pltpu.sync_copy(rows0.at[pl.ds(0, sz)],
                            acc_sp.at[pl.ds(tid * ROWS_PT + off, sz)])
        plsc.subcore_barrier()

    def writeback(dst_hbm):
        plsc.subcore_barrier()
        for off, sz in WB_CHUNKS:
            pltpu.sync_copy(acc_sp.at[pl.ds(tid * ROWS_PT + off, sz)],
                            rows0.at[pl.ds(0, sz)])
            pltpu.sync_copy(rows0.at[pl.ds(0, sz)], dst_hbm.at[pl.ds(off, sz)])
        plsc.subcore_barrier()

    def run(xs_refs, src1, dst2, alpha_t, out_hbm, den_hbm):
        pltpu.sync_copy(src1.at[pl.ds(tid * EPT, EPT)], src_v)
        pltpu.sync_copy(dst2.at[tid], dst_v)

        def gather_issue(c, b, buf):
            pltpu.async_copy(
                xs_refs[c].at[src_v.at[pl.ds(b * BLK, BLK)]], buf, sem_r)

        def alpha_issue(b, buf):
            e0 = tid * EPT + b * BLK
            pltpu.async_copy(
                alpha_t.at[pl.ds(e0 * 16, BLK * 16)], buf, sem_a)

        def drain_rows(buf):
            pltpu.make_async_copy(
                xs_refs[0].at[pl.ds(0, BLK)], buf, sem_r).wait()

        def drain_alpha(buf):
            pltpu.make_async_copy(
                alpha_t.at[pl.ds(0, BLK * 16)], buf, sem_a).wait()

        for c in range(NCHUNK):
            clear_acc()
            gather_issue(c, 0, rows0)
            alpha_issue(0, al0)
            gather_issue(c, 1, rows1)
            alpha_issue(1, al1)

            def scale_block(b, rbuf, abuf):
                def edge(j, _):
                    av = abuf[pl.ds(j * 16, 16)]
                    w0 = av[2 * c]
                    w1 = av[2 * c + 1]
                    for k in range(4):
                        s = pl.ds(k * 16, 16)
                        rbuf[j, s] = rbuf[j, s] * w0
                    for k in range(4, 8):
                        s = pl.ds(k * 16, 16)
                        rbuf[j, s] = rbuf[j, s] * w1
                    return _
                lax.fori_loop(0, BLK, edge, None)
                pltpu.sync_copy(rbuf, acc_sp.at[dst_v.at[b]], add=True)

            def pair(t, _):
                for k in range(2):
                    b = 2 * t + k
                    drain_rows(rows[k])
                    drain_alpha(als[k])
                    scale_block(b, rows[k], als[k])
                    nb = jnp.minimum(b + 2, NBLK - 1)
                    gather_issue(c, nb, rows[k])
                    alpha_issue(nb, als[k])
                return _
            lax.fori_loop(0, NBLK // 2, pair, None)
            drain_rows(rows0)
            drain_alpha(al0)
            scale_block(NBLK - 1, rows0, al0)
            drain_rows(rows1)
            drain_alpha(al1)
            writeback(out_hbm.at[c, pl.ds(tid * ROWS_PT, ROWS_PT)])

        clear_acc()

        def zero_rows1(j, _):
            for k in range(CHUNK // 16):
                rows1[j, pl.ds(k * 16, 16)] = jnp.zeros((16,), jnp.float32)
            return _
        lax.fori_loop(0, BLK, zero_rows1, None)
        alpha_issue(0, al0)
        alpha_issue(1, al1)

        def dblock(b, abuf):
            def edge(j, _):
                rows1[j, pl.ds(0, 16)] = abuf[pl.ds(j * 16, 16)]
                return _
            lax.fori_loop(0, BLK, edge, None)
            pltpu.sync_copy(rows1, acc_sp.at[dst_v.at[b]], add=True)

        def dpair(t, _):
            for k in range(2):
                b = 2 * t + k
                drain_alpha(als[k])
                dblock(b, als[k])
                nb = jnp.minimum(b + 2, NBLK - 1)
                alpha_issue(nb, als[k])
            return _
        lax.fori_loop(0, NBLK // 2, dpair, None)
        drain_alpha(al0)
        dblock(NBLK - 1, al0)
        drain_alpha(al1)
        writeback(den_hbm.at[pl.ds(tid * ROWS_PT, ROWS_PT)])

    @pl.when(core == 0)
    def _():
        run((xp0, xp1, xp2, xp3), src_pl, dst2_pl, alpha_pl,
            out_label, den_pl)

    @pl.when(core == 1)
    def _():
        run((xl0, xl1, xl2, xl3), src_lp, dst2_lp, alpha_lp,
            out_paper, den_lp)


def _att_matrix(att):
    a = jnp.zeros((H, D, CHUNK), jnp.float32)
    a = a.at[jnp.arange(H), :, jnp.arange(H)].set(att)
    return a.reshape(HID, CHUNK)


def _head_expand_matrix():
    r = jnp.zeros((CHUNK, H, D), jnp.float32)
    r = r.at[jnp.arange(H), jnp.arange(H), :].set(1.0)
    return r.reshape(CHUNK, HID)


def kernel(x_paper, x_label, edge_index_pl, edge_index_lp,
           proj_paper_W, proj_paper_b, proj_label_W, proj_label_b,
           att_src_pl, att_dst_pl, att_src_lp, att_dst_lp,
           q, k_W, k_b, lin_W, lin_b):
    del q, k_W, k_b

    a_src_pl_m = _att_matrix(att_src_pl)
    a_dst_pl_m = _att_matrix(att_dst_pl)
    a_src_lp_m = _att_matrix(att_src_lp)
    a_dst_lp_m = _att_matrix(att_dst_lp)
    r_m = _head_expand_matrix()

    bp2 = proj_paper_b.reshape(1, HID)
    bl2 = proj_label_b.reshape(1, HID)
    lb2 = lin_b.reshape(1, EMB)

    xp0, xp1, xp2, xp3, a_src_pl_t, a_dst_lp_t = _k0_call(
        x_paper, proj_paper_W, bp2, a_src_pl_m, a_dst_lp_m)
    xl0, xl1, xl2, xl3, a_dst_pl_t, a_src_lp_t = _k0_call(
        x_label, proj_label_W, bl2, a_dst_pl_m, a_src_lp_m)

    src_pl = edge_index_pl[0]
    dst_pl = edge_index_pl[1]
    src_lp = edge_index_lp[0]
    dst_lp = edge_index_lp[1]
    dst2_pl = dst_pl.reshape(NS, NBLK, BLK)
    dst2_lp = dst_lp.reshape(NS, NBLK, BLK)

    alpha_pl, alpha_lp = _k1(
        a_src_pl_t, a_dst_pl_t, a_src_lp_t, a_dst_lp_t,
        src_pl, dst_pl, src_lp, dst_lp)

    out_label4, den_pl, out_paper4, den_lp = _k2(
        xp0, xp1, xp2, xp3, xl0, xl1, xl2, xl3,
        src_pl, dst2_pl, src_lp, dst2_lp, alpha_pl, alpha_lp)

    h_paper = _k3_call(out_paper4, den_lp, r_m, lin_W, lb2)
    h_label = _k3_call(out_label4, den_pl, r_m, lin_W, lb2)
    return h_paper, h_label


# speedup vs baseline: 8.8028x; 1.0348x over previous
import functools

import jax
import jax.numpy as jnp
from jax import lax
from jax.experimental import pallas as pl
from jax.experimental.pallas import tpu as pltpu
from jax.experimental.pallas import tpu_sc as plsc

N = 10000
E = 160000
HID = 512
H = 8
D = 64
EMB = 256
NC = 2
NS = 16
BLK = 80
EPT = E // NS
NBLK = EPT // BLK
N_PAD = 10112
ROWS_PT = N_PAD // NS
WB_CHUNKS = ((0, 80), (80, 80), (160, 80), (240, 80), (320, 80),
             (400, 80), (480, 80), (560, 72))
RB = 1000
CHUNK = 128
NCHUNK = 4


def _k0_body(x_ref, w_ref, b_ref, a1_ref, a2_ref,
             o0_ref, o1_ref, o2_ref, o3_ref, s1_ref, s2_ref):
    acc = jnp.dot(x_ref[...], w_ref[...], preferred_element_type=jnp.float32)
    acc = acc + b_ref[...]
    outs = (o0_ref, o1_ref, o2_ref, o3_ref)
    for c in range(NCHUNK):
        outs[c][...] = acc[:, c * CHUNK:(c + 1) * CHUNK]
    s1_ref[...] = jnp.dot(acc, a1_ref[...], preferred_element_type=jnp.float32)
    s2_ref[...] = jnp.dot(acc, a2_ref[...], preferred_element_type=jnp.float32)


def _k0_call(x, w, b2, a1, a2):
    grid = (N // RB,)
    return pl.pallas_call(
        _k0_body,
        grid=grid,
        in_specs=[
            pl.BlockSpec((RB, HID), lambda i: (i, 0)),
            pl.BlockSpec((HID, HID), lambda i: (0, 0)),
            pl.BlockSpec((1, HID), lambda i: (0, 0)),
            pl.BlockSpec((HID, CHUNK), lambda i: (0, 0)),
            pl.BlockSpec((HID, CHUNK), lambda i: (0, 0)),
        ],
        out_specs=[pl.BlockSpec((RB, CHUNK), lambda i: (i, 0))] * (NCHUNK + 2),
        out_shape=[jax.ShapeDtypeStruct((N, CHUNK), jnp.float32)] * (NCHUNK + 2),
    )(x, w, b2, a1, a2)


def _k3_body(p_ref, den_ref, r_ref, w_ref, b_ref, o_ref):
    den = jnp.dot(den_ref[...], r_ref[...], preferred_element_type=jnp.float32)
    rec = 1.0 / (den + 1e-16)
    acc = jnp.zeros((RB, EMB), jnp.float32)
    for c in range(NCHUNK):
        u = jnp.maximum(p_ref[c] * rec[:, c * CHUNK:(c + 1) * CHUNK], 0.0)
        wc = w_ref[pl.ds(c * CHUNK, CHUNK), :]
        acc = acc + jnp.dot(u, wc, preferred_element_type=jnp.float32)
    o_ref[...] = acc + b_ref[...]


def _k3_call(p4, den, r, w, b2):
    grid = (N // RB,)
    return pl.pallas_call(
        _k3_body,
        grid=grid,
        in_specs=[
            pl.BlockSpec((NCHUNK, RB, CHUNK), lambda i: (0, i, 0)),
            pl.BlockSpec((RB, CHUNK), lambda i: (i, 0)),
            pl.BlockSpec((CHUNK, HID), lambda i: (0, 0)),
            pl.BlockSpec((HID, EMB), lambda i: (0, 0)),
            pl.BlockSpec((1, EMB), lambda i: (0, 0)),
        ],
        out_specs=pl.BlockSpec((RB, EMB), lambda i: (i, 0)),
        out_shape=jax.ShapeDtypeStruct((N, EMB), jnp.float32),
    )(p4, den, r, w, b2)


_MESH = plsc.VectorSubcoreMesh(core_axis_name="c", subcore_axis_name="s",
                               num_cores=NC, num_subcores=NS)


@functools.partial(
    pl.kernel,
    out_type=[
        jax.ShapeDtypeStruct((E * 16,), jnp.float32),
        jax.ShapeDtypeStruct((E * 16,), jnp.float32),
    ],
    mesh=_MESH,
    scratch_types=[
        pltpu.VMEM((EPT,), jnp.int32),
        pltpu.VMEM((EPT,), jnp.int32),
        pltpu.VMEM((BLK, CHUNK), jnp.float32),
        pltpu.VMEM((BLK, CHUNK), jnp.float32),
        pltpu.VMEM((BLK, CHUNK), jnp.float32),
        pltpu.VMEM((BLK, CHUNK), jnp.float32),
        pltpu.VMEM((BLK * 16,), jnp.float32),
        pltpu.VMEM((BLK * 16,), jnp.float32),
        pltpu.SemaphoreType.DMA,
        pltpu.SemaphoreType.DMA,
        pltpu.SemaphoreType.DMA,
    ],
)
def _k1(a_src_pl, a_dst_pl, a_src_lp, a_dst_lp,
        src_pl, dst_pl, src_lp, dst_lp,
        alpha_pl, alpha_lp,
        src_v, dst_v, as0, as1, ad0, ad1, ex0, ex1, sem_s, sem_d, sem_e):
    tid = lax.axis_index("s")
    core = lax.axis_index("c")
    asrcs = (as0, as1)
    adsts = (ad0, ad1)
    exps = (ex0, ex1)

    def run(a_src_t, a_dst_t, src1, dst1, alpha_out):
        pltpu.sync_copy(src1.at[pl.ds(tid * EPT, EPT)], src_v)
        pltpu.sync_copy(dst1.at[pl.ds(tid * EPT, EPT)], dst_v)

        def issue(b, sbuf, dbuf):
            pltpu.async_copy(
                a_src_t.at[src_v.at[pl.ds(b * BLK, BLK)]], sbuf, sem_s)
            pltpu.async_copy(
                a_dst_t.at[dst_v.at[pl.ds(b * BLK, BLK)]], dbuf, sem_d)

        def drain_gathers(sbuf, dbuf):
            pltpu.make_async_copy(
                a_src_t.at[pl.ds(0, BLK)], sbuf, sem_s).wait()
            pltpu.make_async_copy(
                a_dst_t.at[pl.ds(0, BLK)], dbuf, sem_d).wait()

        def drain_store():
            pltpu.make_async_copy(
                alpha_out.at[pl.ds(0, BLK * 16)], ex0, sem_e).wait()

        def proc(b, sbuf, dbuf, ebuf):
            def edge2(t, _):
                for u in range(2):
                    j = 2 * t + u
                    v = sbuf[j, pl.ds(0, 16)] + dbuf[j, pl.ds(0, 16)]
                    v = jnp.where(v >= 0.0, v, 0.2 * v)
                    ebuf[pl.ds(j * 16, 16)] = jnp.exp(v)
                return _
            lax.fori_loop(0, BLK // 2, edge2, None)
            e0 = tid * EPT + b * BLK
            pltpu.async_copy(
                ebuf, alpha_out.at[pl.ds(e0 * 16, BLK * 16)], sem_e)

        issue(0, as0, ad0)
        issue(1, as1, ad1)

        def pair(t, _):
            for k in range(2):
                b = 2 * t + k
                drain_gathers(asrcs[k], adsts[k])

                @pl.when(t > 0)
                def _():
                    drain_store()
                proc(b, asrcs[k], adsts[k], exps[k])
                nb = jnp.minimum(b + 2, NBLK - 1)
                issue(nb, asrcs[k], adsts[k])
            return _
        lax.fori_loop(0, NBLK // 2, pair, None)
        drain_gathers(as0, ad0)
        drain_store()
        proc(NBLK - 1, as0, ad0, ex0)
        drain_gathers(as1, ad1)
        drain_store()
        drain_store()

    @pl.when(core == 0)
    def _():
        run(a_src_pl, a_dst_pl, src_pl, dst_pl, alpha_pl)

    @pl.when(core == 1)
    def _():
        run(a_src_lp, a_dst_lp, src_lp, dst_lp, alpha_lp)


@functools.partial(
    pl.kernel,
    out_type=[
        jax.ShapeDtypeStruct((NCHUNK, N_PAD, CHUNK), jnp.float32),
        jax.ShapeDtypeStruct((N_PAD, CHUNK), jnp.float32),
        jax.ShapeDtypeStruct((NCHUNK, N_PAD, CHUNK), jnp.float32),
        jax.ShapeDtypeStruct((N_PAD, CHUNK), jnp.float32),
    ],
    mesh=_MESH,
    scratch_types=[
        pltpu.VMEM_SHARED((N_PAD, CHUNK), jnp.float32),
        pltpu.VMEM((EPT,), jnp.int32),
        pltpu.VMEM((NBLK, BLK), jnp.int32),
        pltpu.VMEM((BLK * 16,), jnp.float32),
        pltpu.VMEM((BLK * 16,), jnp.float32),
        pltpu.VMEM((BLK, CHUNK), jnp.float32),
        pltpu.VMEM((BLK, CHUNK), jnp.float32),
        pltpu.SemaphoreType.DMA,
        pltpu.SemaphoreType.DMA,
    ],
)
def _k2(xp0, xp1, xp2, xp3, xl0, xl1, xl2, xl3,
        src_pl, dst2_pl, src_lp, dst2_lp, alpha_pl, alpha_lp,
        out_label, den_pl, out_paper, den_lp,
        acc_sp, src_v, dst_v, al0, al1, rows0, rows1, sem_r, sem_a):
    tid = lax.axis_index("s")
    core = lax.axis_index("c")
    rows = (rows0, rows1)
    als = (al0, al1)

    def zero_rows0(j, _):
        for k in range(CHUNK // 16):
            rows0[j, pl.ds(k * 16, 16)] = jnp.zeros((16,), jnp.float32)
        return _

    def clear_acc():
        lax.fori_loop(0, BLK, zero_rows0, None)
        for off, sz in WB_CHUNKS:
            pltpu.sync_copy(rows0.at[pl.ds(0, sz)],
                            acc_sp.at[pl.ds(tid * ROWS_PT + off, sz)])
        plsc.subcore_barrier()

    def writeback(dst_hbm):
        plsc.subcore_barrier()
        for off, sz in WB_CHUNKS:
            pltpu.sync_copy(acc_sp.at[pl.ds(tid * ROWS_PT + off, sz)],
                            rows0.at[pl.ds(0, sz)])
            pltpu.sync_copy(rows0.at[pl.ds(0, sz)], dst_hbm.at[pl.ds(off, sz)])
        plsc.subcore_barrier()

    def run(xs_refs, src1, dst2, alpha_t, out_hbm, den_hbm):
        pltpu.sync_copy(src1.at[pl.ds(tid * EPT, EPT)], src_v)
        pltpu.sync_copy(dst2.at[tid], dst_v)

        def gather_issue(c, b, buf):
            pltpu.async_copy(
                xs_refs[c].at[src_v.at[pl.ds(b * BLK, BLK)]], buf, sem_r)

        def alpha_issue(b, buf):
            e0 = tid * EPT + b * BLK
            pltpu.async_copy(
                alpha_t.at[pl.ds(e0 * 16, BLK * 16)], buf, sem_a)

        def drain_rows(buf):
            pltpu.make_async_copy(
                xs_refs[0].at[pl.ds(0, BLK)], buf, sem_r).wait()

        def drain_alpha(buf):
            pltpu.make_async_copy(
                alpha_t.at[pl.ds(0, BLK * 16)], buf, sem_a).wait()

        for c in range(NCHUNK):
            clear_acc()
            gather_issue(c, 0, rows0)
            alpha_issue(0, al0)
            gather_issue(c, 1, rows1)
            alpha_issue(1, al1)

            def scale_block(b, rbuf, abuf):
                def edge2(t, _):
                    for u in range(2):
                        j = 2 * t + u
                        av = abuf[pl.ds(j * 16, 16)]
                        w0 = av[2 * c]
                        w1 = av[2 * c + 1]
                        for k in range(4):
                            s = pl.ds(k * 16, 16)
                            rbuf[j, s] = rbuf[j, s] * w0
                        for k in range(4, 8):
                            s = pl.ds(k * 16, 16)
                            rbuf[j, s] = rbuf[j, s] * w1
                    return _
                lax.fori_loop(0, BLK // 2, edge2, None)
                pltpu.sync_copy(rbuf, acc_sp.at[dst_v.at[b]], add=True)

            def pair(t, _):
                for k in range(2):
                    b = 2 * t + k
                    drain_rows(rows[k])
                    drain_alpha(als[k])
                    scale_block(b, rows[k], als[k])
                    nb = jnp.minimum(b + 2, NBLK - 1)
                    gather_issue(c, nb, rows[k])
                    alpha_issue(nb, als[k])
                return _
            lax.fori_loop(0, NBLK // 2, pair, None)
            drain_rows(rows0)
            drain_alpha(al0)
            scale_block(NBLK - 1, rows0, al0)
            drain_rows(rows1)
            drain_alpha(al1)
            writeback(out_hbm.at[c, pl.ds(tid * ROWS_PT, ROWS_PT)])

        clear_acc()

        def zero_rows1(j, _):
            for k in range(CHUNK // 16):
                rows1[j, pl.ds(k * 16, 16)] = jnp.zeros((16,), jnp.float32)
            return _
        lax.fori_loop(0, BLK, zero_rows1, None)
        alpha_issue(0, al0)
        alpha_issue(1, al1)

        def dblock(b, abuf):
            def edge4(t, _):
                for u in range(4):
                    j = 4 * t + u
                    rows1[j, pl.ds(0, 16)] = abuf[pl.ds(j * 16, 16)]
                return _
            lax.fori_loop(0, BLK // 4, edge4, None)
            pltpu.sync_copy(rows1, acc_sp.at[dst_v.at[b]], add=True)

        def dpair(t, _):
            for k in range(2):
                b = 2 * t + k
                drain_alpha(als[k])
                dblock(b, als[k])
                nb = jnp.minimum(b + 2, NBLK - 1)
                alpha_issue(nb, als[k])
            return _
        lax.fori_loop(0, NBLK // 2, dpair, None)
        drain_alpha(al0)
        dblock(NBLK - 1, al0)
        drain_alpha(al1)
        writeback(den_hbm.at[pl.ds(tid * ROWS_PT, ROWS_PT)])

    @pl.when(core == 0)
    def _():
        run((xp0, xp1, xp2, xp3), src_pl, dst2_pl, alpha_pl,
            out_label, den_pl)

    @pl.when(core == 1)
    def _():
        run((xl0, xl1, xl2, xl3), src_lp, dst2_lp, alpha_lp,
            out_paper, den_lp)


def _att_matrix(att):
    a = jnp.zeros((H, D, CHUNK), jnp.float32)
    a = a.at[jnp.arange(H), :, jnp.arange(H)].set(att)
    return a.reshape(HID, CHUNK)


def _head_expand_matrix():
    r = jnp.zeros((CHUNK, H, D), jnp.float32)
    r = r.at[jnp.arange(H), jnp.arange(H), :].set(1.0)
    return r.reshape(CHUNK, HID)


def kernel(x_paper, x_label, edge_index_pl, edge_index_lp,
           proj_paper_W, proj_paper_b, proj_label_W, proj_label_b,
           att_src_pl, att_dst_pl, att_src_lp, att_dst_lp,
           q, k_W, k_b, lin_W, lin_b):
    del q, k_W, k_b

    a_src_pl_m = _att_matrix(att_src_pl)
    a_dst_pl_m = _att_matrix(att_dst_pl)
    a_src_lp_m = _att_matrix(att_src_lp)
    a_dst_lp_m = _att_matrix(att_dst_lp)
    r_m = _head_expand_matrix()

    bp2 = proj_paper_b.reshape(1, HID)
    bl2 = proj_label_b.reshape(1, HID)
    lb2 = lin_b.reshape(1, EMB)

    xp0, xp1, xp2, xp3, a_src_pl_t, a_dst_lp_t = _k0_call(
        x_paper, proj_paper_W, bp2, a_src_pl_m, a_dst_lp_m)
    xl0, xl1, xl2, xl3, a_dst_pl_t, a_src_lp_t = _k0_call(
        x_label, proj_label_W, bl2, a_dst_pl_m, a_src_lp_m)

    src_pl = edge_index_pl[0]
    dst_pl = edge_index_pl[1]
    src_lp = edge_index_lp[0]
    dst_lp = edge_index_lp[1]
    dst2_pl = dst_pl.reshape(NS, NBLK, BLK)
    dst2_lp = dst_lp.reshape(NS, NBLK, BLK)

    alpha_pl, alpha_lp = _k1(
        a_src_pl_t, a_dst_pl_t, a_src_lp_t, a_dst_lp_t,
        src_pl, dst_pl, src_lp, dst_lp)

    out_label4, den_pl, out_paper4, den_lp = _k2(
        xp0, xp1, xp2, xp3, xl0, xl1, xl2, xl3,
        src_pl, dst2_pl, src_lp, dst2_lp, alpha_pl, alpha_lp)

    h_paper = _k3_call(out_paper4, den_lp, r_m, lin_W, lb2)
    h_label = _k3_call(out_label4, den_pl, r_m, lin_W, lb2)
    return h_paper, h_label
